# baseline (speedup 1.0000x reference)
import math

import jax
import jax.numpy as jnp
from jax import lax
from jax.experimental import pallas as pl
from jax.experimental.pallas import tpu as pltpu
from jax.experimental.pallas import tpu_sc as plsc

_LANES = 16
_CH = 16
_NBT = 3


def _scale_tc(x, scale):
    B, S, D = x.shape
    xs = x.reshape(B * S, D)
    rows = B * S
    blk = 1024

    def body(x_ref, o_ref):
        o_ref[...] = x_ref[...] * scale

    out = pl.pallas_call(
        body,
        out_shape=jax.ShapeDtypeStruct((rows, D), jnp.float32),
        grid=(rows // blk,),
        in_specs=[pl.BlockSpec((blk, D), lambda i: (i, 0))],
        out_specs=pl.BlockSpec((blk, D), lambda i: (i, 0)),
    )(xs)
    return out.reshape(B, S, D)


def _pos_emb_sc(pe2d, pq_flat, B, S, D):
    info = plsc.get_sparse_core_info()
    NC, NS = info.num_cores, info.num_subcores
    bpc = B // NC
    rpw = S // NS
    nch = rpw // _CH
    ngrp = D // _LANES

    mesh = plsc.VectorSubcoreMesh(core_axis_name="c", subcore_axis_name="s")

    def body(pe_hbm, pq_hbm, out_hbm, pq_v, tbuf, obuf, *sems):
        ld_sems = sems[:_NBT]
        st_sems = sems[_NBT:]
        c = lax.axis_index("c")
        s = lax.axis_index("s")
        trow = pl.multiple_of(s * rpw, 8)
        orow = [
            pl.multiple_of((c * bpc + b2) * S + s * rpw, 8) for b2 in range(bpc)
        ]

        pltpu.sync_copy(pq_hbm.at[pl.ds(c * (bpc * 2 * D), bpc * 2 * D)], pq_v)

        lanes = lax.iota(jnp.int32, _LANES)
        swapidx = jnp.bitwise_xor(lanes, 1)[:, None]
        gdn = lax.GatherDimensionNumbers(
            offset_dims=(), collapsed_slice_dims=(0,), start_index_map=(0,)
        )

        def swap(t):
            return lax.gather(
                t, swapidx, gdn, (1,),
                mode=lax.GatherScatterMode.PROMISE_IN_BOUNDS,
            )

        def compute(pt, po):
            tb = tbuf.at[pt]

            def gbody(g, _):
                g0 = g * _LANES
                pq = [
                    (
                        pq_v[pl.ds(b2 * 2 * D + g0, _LANES)],
                        pq_v[pl.ds(b2 * 2 * D + D + g0, _LANES)],
                    )
                    for b2 in range(bpc)
                ]
                for i in range(_CH):
                    t = tb[i, pl.ds(g0, _LANES)]
                    tsw = swap(t)
                    for b2 in range(bpc):
                        p, q = pq[b2]
                        obuf[b2, po, i, pl.ds(g0, _LANES)] = t * p + tsw * q
                return 0

            lax.fori_loop(0, ngrp, gbody, 0)

        ld = [None] * nch
        st = [[None] * nch for _ in range(bpc)]

        def issue_ld(j):
            pt = j % _NBT
            ld[j] = pltpu.async_copy(
                pe_hbm.at[pl.ds(trow + j * _CH, _CH), :], tbuf.at[pt], ld_sems[pt]
            )

        for j in range(min(2, nch)):
            issue_ld(j)
        for j in range(nch):
            pt = j % _NBT
            po = j % 2
            ld[j].wait()
            if j + 2 < nch:
                issue_ld(j + 2)
            if j >= 2:
                for b2 in range(bpc):
                    st[b2][j - 2].wait()
            compute(pt, po)
            for b2 in range(bpc):
                st[b2][j] = pltpu.async_copy(
                    obuf.at[b2, po],
                    out_hbm.at[pl.ds(orow[b2] + j * _CH, _CH), :],
                    st_sems[b2 * 2 + po],
                )
        for j in range(max(0, nch - 2), nch):
            for b2 in range(bpc):
                st[b2][j].wait()

    return pl.kernel(
        body,
        out_type=jax.ShapeDtypeStruct((B * S, D), jnp.float32),
        mesh=mesh,
        scratch_types=[
            pltpu.VMEM((2 * 2 * D,), jnp.float32),
            pltpu.VMEM((_NBT, _CH, D), jnp.float32),
            pltpu.VMEM((2, 2, _CH, D), jnp.float32),
        ]
        + [pltpu.SemaphoreType.DMA] * (_NBT + 4),
        compiler_params=pltpu.CompilerParams(needs_layout_passes=False),
    )(pe2d, pq_flat)


def kernel(x, offset, pe):
    B, S, D = x.shape
    scale = math.sqrt(D)
    pe2d = pe[0]

    rows = jnp.take(pe2d, offset, axis=0)
    k = jnp.arange(D, dtype=jnp.int32)
    p_mat = jnp.take(rows, jnp.bitwise_or(k, 1), axis=1)
    sign = jnp.where(k % 2 == 0, 1.0, -1.0).astype(jnp.float32)
    q_mat = jnp.take(rows, jnp.bitwise_and(k, ~1), axis=1) * sign
    pq_flat = jnp.stack([p_mat, q_mat], axis=1).reshape(-1)

    pos_emb = _pos_emb_sc(pe2d, pq_flat, B, S, D).reshape(B, S, D)
    x_scaled = _scale_tc(x, scale)
    return (x_scaled, pos_emb)


# speedup vs baseline: 1.2459x; 1.2459x over previous
import math

import jax
import jax.numpy as jnp
from jax import lax
from jax.experimental import pallas as pl
from jax.experimental.pallas import tpu as pltpu
from jax.experimental.pallas import tpu_sc as plsc

_LANES = 16
_CH = 16
_NBT = 3


def _scale_tc(x, scale):
    B, S, D = x.shape
    xs = x.reshape(B * S, D)
    rows = B * S
    blk = 1024

    def body(x_ref, o_ref):
        o_ref[...] = x_ref[...] * scale

    out = pl.pallas_call(
        body,
        out_shape=jax.ShapeDtypeStruct((rows, D), jnp.float32),
        grid=(rows // blk,),
        in_specs=[pl.BlockSpec((blk, D), lambda i: (i, 0))],
        out_specs=pl.BlockSpec((blk, D), lambda i: (i, 0)),
    )(xs)
    return out.reshape(B, S, D)


def _pos_emb_sc(pe2d, offset, B, S, D):
    info = plsc.get_sparse_core_info()
    NC, NS = info.num_cores, info.num_subcores
    bpc = B // NC
    rpw = S // NS
    nch = rpw // _CH
    ngrp = D // _LANES

    mesh = plsc.VectorSubcoreMesh(core_axis_name="c", subcore_axis_name="s")

    def body(pe_hbm, off_hbm, out_hbm, off_v, idx_v, prow_v, pq_v, tbuf, obuf,
             *sems):
        gsem = sems[0]
        ld_sems = sems[1:1 + _NBT]
        st_sems = sems[1 + _NBT:]
        c = lax.axis_index("c")
        s = lax.axis_index("s")
        trow = pl.multiple_of(s * rpw, 8)
        orow = [
            pl.multiple_of((c * bpc + b2) * S + s * rpw, 8) for b2 in range(bpc)
        ]

        lanes = lax.iota(jnp.int32, _LANES)
        swapidx = jnp.bitwise_xor(lanes, 1)[:, None]
        gdn = lax.GatherDimensionNumbers(
            offset_dims=(), collapsed_slice_dims=(0,), start_index_map=(0,)
        )

        def vperm(t, idx2d):
            return lax.gather(
                t, idx2d, gdn, (1,),
                mode=lax.GatherScatterMode.PROMISE_IN_BOUNDS,
            )

        pltpu.sync_copy(off_hbm, off_v)
        bidx = c * bpc + jnp.bitwise_and(lanes, bpc - 1)
        idx_v[...] = plsc.load_gather(off_v, [bidx])
        pltpu.async_copy(
            pe_hbm.at[idx_v.at[pl.ds(0, bpc)]], prow_v, gsem
        ).wait()
        oridx = jnp.bitwise_or(lanes, 1)[:, None]
        andidx = jnp.bitwise_and(lanes, ~1)[:, None]
        signv = jnp.where(jnp.bitwise_and(lanes, 1) == 0, 1.0, -1.0).astype(
            jnp.float32
        )
        for b2 in range(bpc):
            for g in range(ngrp):
                g0 = g * _LANES
                v = prow_v[b2, pl.ds(g0, _LANES)]
                pq_v[pl.ds(b2 * 2 * D + g0, _LANES)] = vperm(v, oridx)
                pq_v[pl.ds(b2 * 2 * D + D + g0, _LANES)] = (
                    vperm(v, andidx) * signv
                )

        def compute(pt, po):
            tb = tbuf.at[pt]

            @plsc.parallel_loop(0, ngrp, unroll=4)
            def gbody(g):
                g0 = g * _LANES
                pq = [
                    (
                        pq_v[pl.ds(b2 * 2 * D + g0, _LANES)],
                        pq_v[pl.ds(b2 * 2 * D + D + g0, _LANES)],
                    )
                    for b2 in range(bpc)
                ]
                for i in range(_CH):
                    t = tb[i, pl.ds(g0, _LANES)]
                    tsw = vperm(t, swapidx)
                    for b2 in range(bpc):
                        p, q = pq[b2]
                        obuf[b2, po, i, pl.ds(g0, _LANES)] = t * p + tsw * q

        ld = [None] * nch
        st = [[None] * nch for _ in range(bpc)]

        def issue_ld(j):
            pt = j % _NBT
            ld[j] = pltpu.async_copy(
                pe_hbm.at[pl.ds(trow + j * _CH, _CH), :], tbuf.at[pt], ld_sems[pt]
            )

        for j in range(min(2, nch)):
            issue_ld(j)
        for j in range(nch):
            pt = j % _NBT
            po = j % 2
            ld[j].wait()
            if j + 2 < nch:
                issue_ld(j + 2)
            if j >= 2:
                for b2 in range(bpc):
                    st[b2][j - 2].wait()
            compute(pt, po)
            for b2 in range(bpc):
                st[b2][j] = pltpu.async_copy(
                    obuf.at[b2, po],
                    out_hbm.at[pl.ds(orow[b2] + j * _CH, _CH), :],
                    st_sems[b2 * 2 + po],
                )
        for j in range(max(0, nch - 2), nch):
            for b2 in range(bpc):
                st[b2][j].wait()

    return pl.kernel(
        body,
        out_type=jax.ShapeDtypeStruct((B * S, D), jnp.float32),
        mesh=mesh,
        scratch_types=[
            pltpu.VMEM((B,), jnp.int32),
            pltpu.VMEM((_LANES,), jnp.int32),
            pltpu.VMEM((2, D), jnp.float32),
            pltpu.VMEM((2 * 2 * D,), jnp.float32),
            pltpu.VMEM((_NBT, _CH, D), jnp.float32),
            pltpu.VMEM((2, 2, _CH, D), jnp.float32),
        ]
        + [pltpu.SemaphoreType.DMA] * (1 + _NBT + 4),
        compiler_params=pltpu.CompilerParams(needs_layout_passes=False),
    )(pe2d, offset)


def kernel(x, offset, pe):
    B, S, D = x.shape
    scale = math.sqrt(D)
    pe2d = pe[0]
    pos_emb = _pos_emb_sc(pe2d, offset, B, S, D).reshape(B, S, D)
    x_scaled = _scale_tc(x, scale)
    return (x_scaled, pos_emb)


# speedup vs baseline: 1.2736x; 1.0223x over previous
import math

import jax
import jax.numpy as jnp
from jax import lax
from jax.experimental import pallas as pl
from jax.experimental.pallas import tpu as pltpu
from jax.experimental.pallas import tpu_sc as plsc

_LANES = 16
_CH = 16
_NBT = 3


def _scale_tc(x, scale):
    B, S, D = x.shape
    xs = x.reshape(B * S, D)
    rows = B * S
    blk = 1024

    def body(x_ref, o_ref):
        o_ref[...] = x_ref[...] * scale

    out = pl.pallas_call(
        body,
        out_shape=jax.ShapeDtypeStruct((rows, D), jnp.float32),
        grid=(rows // blk,),
        in_specs=[pl.BlockSpec((blk, D), lambda i: (i, 0))],
        out_specs=pl.BlockSpec((blk, D), lambda i: (i, 0)),
    )(xs)
    return out.reshape(B, S, D)


def _pos_emb_sc(pe2d, offset, B, S, D):
    info = plsc.get_sparse_core_info()
    NC, NS = info.num_cores, info.num_subcores
    bpc = B // NC
    rpw = S // NS
    nch = rpw // _CH
    ngrp = D // _LANES

    mesh = plsc.VectorSubcoreMesh(core_axis_name="c", subcore_axis_name="s")

    def body(pe_hbm, off_hbm, out_hbm, off_v, idx_v, prow_v, pq_v, tbuf, obuf,
             *sems):
        gsem = sems[0]
        ld_sems = sems[1:1 + _NBT]
        st_sems = sems[1 + _NBT:]
        c = lax.axis_index("c")
        s = lax.axis_index("s")
        trow = pl.multiple_of(s * rpw, 8)
        orow = [
            pl.multiple_of((c * bpc + b2) * S + s * rpw, 8) for b2 in range(bpc)
        ]

        lanes = lax.iota(jnp.int32, _LANES)
        swapidx = jnp.bitwise_xor(lanes, 1)[:, None]
        gdn = lax.GatherDimensionNumbers(
            offset_dims=(), collapsed_slice_dims=(0,), start_index_map=(0,)
        )

        def vperm(t, idx2d):
            return lax.gather(
                t, idx2d, gdn, (1,),
                mode=lax.GatherScatterMode.PROMISE_IN_BOUNDS,
            )

        pltpu.sync_copy(off_hbm, off_v)
        bidx = c * bpc + jnp.bitwise_and(lanes, bpc - 1)
        idx_v[...] = plsc.load_gather(off_v, [bidx])
        pltpu.async_copy(
            pe_hbm.at[idx_v.at[pl.ds(0, bpc)]], prow_v, gsem
        ).wait()
        oridx = jnp.bitwise_or(lanes, 1)[:, None]
        andidx = jnp.bitwise_and(lanes, ~1)[:, None]
        signv = jnp.where(jnp.bitwise_and(lanes, 1) == 0, 1.0, -1.0).astype(
            jnp.float32
        )
        @plsc.parallel_loop(0, ngrp)
        def _pqprep(g):
            g0 = g * _LANES
            for b2 in range(bpc):
                v = prow_v[b2, pl.ds(g0, _LANES)]
                pq_v[pl.ds(b2 * 2 * D + g0, _LANES)] = vperm(v, oridx)
                pq_v[pl.ds(b2 * 2 * D + D + g0, _LANES)] = (
                    vperm(v, andidx) * signv
                )

        def compute(pt, po):
            tb = tbuf.at[pt]

            @plsc.parallel_loop(0, ngrp, unroll=2)
            def gbody(g):
                g0 = g * _LANES
                pq = [
                    (
                        pq_v[pl.ds(b2 * 2 * D + g0, _LANES)],
                        pq_v[pl.ds(b2 * 2 * D + D + g0, _LANES)],
                    )
                    for b2 in range(bpc)
                ]
                for i in range(_CH):
                    t = tb[i, pl.ds(g0, _LANES)]
                    tsw = vperm(t, swapidx)
                    for b2 in range(bpc):
                        p, q = pq[b2]
                        obuf[b2, po, i, pl.ds(g0, _LANES)] = t * p + tsw * q

        ld = [None] * nch
        st = [[None] * nch for _ in range(bpc)]

        def issue_ld(j):
            pt = j % _NBT
            ld[j] = pltpu.async_copy(
                pe_hbm.at[pl.ds(trow + j * _CH, _CH), :], tbuf.at[pt], ld_sems[pt]
            )

        for j in range(min(2, nch)):
            issue_ld(j)
        for j in range(nch):
            pt = j % _NBT
            po = j % 2
            ld[j].wait()
            if j + 2 < nch:
                issue_ld(j + 2)
            if j >= 2:
                for b2 in range(bpc):
                    st[b2][j - 2].wait()
            compute(pt, po)
            for b2 in range(bpc):
                st[b2][j] = pltpu.async_copy(
                    obuf.at[b2, po],
                    out_hbm.at[pl.ds(orow[b2] + j * _CH, _CH), :],
                    st_sems[b2 * 2 + po],
                )
        for j in range(max(0, nch - 2), nch):
            for b2 in range(bpc):
                st[b2][j].wait()

    return pl.kernel(
        body,
        out_type=jax.ShapeDtypeStruct((B * S, D), jnp.float32),
        mesh=mesh,
        scratch_types=[
            pltpu.VMEM((B,), jnp.int32),
            pltpu.VMEM((_LANES,), jnp.int32),
            pltpu.VMEM((2, D), jnp.float32),
            pltpu.VMEM((2 * 2 * D,), jnp.float32),
            pltpu.VMEM((_NBT, _CH, D), jnp.float32),
            pltpu.VMEM((2, 2, _CH, D), jnp.float32),
        ]
        + [pltpu.SemaphoreType.DMA] * (1 + _NBT + 4),
        compiler_params=pltpu.CompilerParams(needs_layout_passes=False),
    )(pe2d, offset)


def kernel(x, offset, pe):
    B, S, D = x.shape
    scale = math.sqrt(D)
    pe2d = pe[0]
    pos_emb = _pos_emb_sc(pe2d, offset, B, S, D).reshape(B, S, D)
    x_scaled = _scale_tc(x, scale)
    return (x_scaled, pos_emb)


# speedup vs baseline: 1.3505x; 1.0604x over previous
import math

import jax
import jax.numpy as jnp
from jax import lax
from jax.experimental import pallas as pl
from jax.experimental.pallas import tpu as pltpu
from jax.experimental.pallas import tpu_sc as plsc

_LANES = 16
_CH = 8
_NBT = 3


def _scale_tc(x, scale):
    B, S, D = x.shape
    xs = x.reshape(B * S, D)
    rows = B * S
    blk = 1024

    def body(x_ref, o_ref):
        o_ref[...] = x_ref[...] * scale

    out = pl.pallas_call(
        body,
        out_shape=jax.ShapeDtypeStruct((rows, D), jnp.float32),
        grid=(rows // blk,),
        in_specs=[pl.BlockSpec((blk, D), lambda i: (i, 0))],
        out_specs=pl.BlockSpec((blk, D), lambda i: (i, 0)),
    )(xs)
    return out.reshape(B, S, D)


def _pos_emb_sc(pe2d, offset, B, S, D):
    info = plsc.get_sparse_core_info()
    NC, NS = info.num_cores, info.num_subcores
    bpc = B
    rpw = S // (NS * NC)
    nch = rpw // _CH
    ngrp = D // _LANES

    mesh = plsc.VectorSubcoreMesh(core_axis_name="c", subcore_axis_name="s")

    def body(pe_hbm, off_hbm, out_hbm, off_v, idx_v, prow_v, pq_v, tbuf, obuf,
             *sems):
        gsem = sems[0]
        ld_sems = sems[1:1 + _NBT]
        st_sems = sems[1 + _NBT:]
        c = lax.axis_index("c")
        s = lax.axis_index("s")
        wrow = c * (S // NC) + s * rpw
        trow = pl.multiple_of(wrow, 8)
        orow = [pl.multiple_of(b2 * S + wrow, 8) for b2 in range(bpc)]

        lanes = lax.iota(jnp.int32, _LANES)
        swapidx = jnp.bitwise_xor(lanes, 1)[:, None]
        gdn = lax.GatherDimensionNumbers(
            offset_dims=(), collapsed_slice_dims=(0,), start_index_map=(0,)
        )

        def vperm(t, idx2d):
            return lax.gather(
                t, idx2d, gdn, (1,),
                mode=lax.GatherScatterMode.PROMISE_IN_BOUNDS,
            )

        pltpu.sync_copy(off_hbm, off_v)
        bidx = jnp.bitwise_and(lanes, bpc - 1)
        idx_v[...] = plsc.load_gather(off_v, [bidx])
        pltpu.async_copy(
            pe_hbm.at[idx_v.at[pl.ds(0, bpc)]], prow_v, gsem
        ).wait()
        oridx = jnp.bitwise_or(lanes, 1)[:, None]
        andidx = jnp.bitwise_and(lanes, ~1)[:, None]
        signv = jnp.where(jnp.bitwise_and(lanes, 1) == 0, 1.0, -1.0).astype(
            jnp.float32
        )
        @plsc.parallel_loop(0, ngrp)
        def _pqprep(g):
            g0 = g * _LANES
            for b2 in range(bpc):
                v = prow_v[b2, pl.ds(g0, _LANES)]
                pq_v[pl.ds(b2 * 2 * D + g0, _LANES)] = vperm(v, oridx)
                pq_v[pl.ds(b2 * 2 * D + D + g0, _LANES)] = (
                    vperm(v, andidx) * signv
                )

        def compute(pt, po):
            tb = tbuf.at[pt]

            @plsc.parallel_loop(0, ngrp, unroll=2)
            def gbody(g):
                g0 = g * _LANES
                pq = [
                    (
                        pq_v[pl.ds(b2 * 2 * D + g0, _LANES)],
                        pq_v[pl.ds(b2 * 2 * D + D + g0, _LANES)],
                    )
                    for b2 in range(bpc)
                ]
                for i in range(_CH):
                    t = tb[i, pl.ds(g0, _LANES)]
                    tsw = vperm(t, swapidx)
                    for b2 in range(bpc):
                        p, q = pq[b2]
                        obuf[b2, po, i, pl.ds(g0, _LANES)] = t * p + tsw * q

        ld = [None] * nch
        st = [[None] * nch for _ in range(bpc)]

        def issue_ld(j):
            pt = j % _NBT
            ld[j] = pltpu.async_copy(
                pe_hbm.at[pl.ds(trow + j * _CH, _CH), :], tbuf.at[pt], ld_sems[pt]
            )

        for j in range(min(2, nch)):
            issue_ld(j)
        for j in range(nch):
            pt = j % _NBT
            po = j % 2
            ld[j].wait()
            if j + 2 < nch:
                issue_ld(j + 2)
            if j >= 2:
                for b2 in range(bpc):
                    st[b2][j - 2].wait()
            compute(pt, po)
            for b2 in range(bpc):
                st[b2][j] = pltpu.async_copy(
                    obuf.at[b2, po],
                    out_hbm.at[pl.ds(orow[b2] + j * _CH, _CH), :],
                    st_sems[b2 * 2 + po],
                )
        for j in range(max(0, nch - 2), nch):
            for b2 in range(bpc):
                st[b2][j].wait()

    return pl.kernel(
        body,
        out_type=jax.ShapeDtypeStruct((B * S, D), jnp.float32),
        mesh=mesh,
        scratch_types=[
            pltpu.VMEM((B,), jnp.int32),
            pltpu.VMEM((_LANES,), jnp.int32),
            pltpu.VMEM((B, D), jnp.float32),
            pltpu.VMEM((B * 2 * D,), jnp.float32),
            pltpu.VMEM((_NBT, _CH, D), jnp.float32),
            pltpu.VMEM((B, 2, _CH, D), jnp.float32),
        ]
        + [pltpu.SemaphoreType.DMA] * (1 + _NBT + 2 * B),
        compiler_params=pltpu.CompilerParams(needs_layout_passes=False),
    )(pe2d, offset)


def kernel(x, offset, pe):
    B, S, D = x.shape
    scale = math.sqrt(D)
    pe2d = pe[0]
    pos_emb = _pos_emb_sc(pe2d, offset, B, S, D).reshape(B, S, D)
    x_scaled = _scale_tc(x, scale)
    return (x_scaled, pos_emb)


# speedup vs baseline: 1.3831x; 1.0241x over previous
import math

import jax
import jax.numpy as jnp
from jax import lax
from jax.experimental import pallas as pl
from jax.experimental.pallas import tpu as pltpu
from jax.experimental.pallas import tpu_sc as plsc

_LANES = 16
_CH = 8
_NBT = 3


def _scale_tc(x, scale):
    B, S, D = x.shape
    xs = x.reshape(B * S, D)
    rows = B * S
    blk = 1024

    def body(x_ref, o_ref):
        o_ref[...] = x_ref[...] * scale

    out = pl.pallas_call(
        body,
        out_shape=jax.ShapeDtypeStruct((rows, D), jnp.float32),
        grid=(rows // blk,),
        in_specs=[pl.BlockSpec((blk, D), lambda i: (i, 0))],
        out_specs=pl.BlockSpec((blk, D), lambda i: (i, 0)),
    )(xs)
    return out.reshape(B, S, D)


def _pos_emb_sc(pe2d, offset, B, S, D):
    info = plsc.get_sparse_core_info()
    NC, NS = info.num_cores, info.num_subcores
    bpc = B
    rpw = S // (NS * NC)
    nch = rpw // _CH
    ngrp = D // _LANES

    mesh = plsc.VectorSubcoreMesh(core_axis_name="c", subcore_axis_name="s")

    def body(pe_hbm, off_hbm, out_hbm, off_v, idx_v, prow_v, pq_v, tbuf, obuf,
             *sems):
        gsem = sems[0]
        ld_sems = sems[1:1 + _NBT]
        st_sems = sems[1 + _NBT:]
        c = lax.axis_index("c")
        s = lax.axis_index("s")
        wrow = c * (S // NC) + s * rpw
        trow = pl.multiple_of(wrow, 8)
        orow = [pl.multiple_of(b2 * S + wrow, 8) for b2 in range(bpc)]

        lanes = lax.iota(jnp.int32, _LANES)
        swapidx = jnp.bitwise_xor(lanes, 1)[:, None]
        gdn = lax.GatherDimensionNumbers(
            offset_dims=(), collapsed_slice_dims=(0,), start_index_map=(0,)
        )

        def vperm(t, idx2d):
            return lax.gather(
                t, idx2d, gdn, (1,),
                mode=lax.GatherScatterMode.PROMISE_IN_BOUNDS,
            )

        pltpu.sync_copy(off_hbm, off_v)
        bidx = jnp.bitwise_and(lanes, bpc - 1)
        idx_v[...] = plsc.load_gather(off_v, [bidx])
        pq_gather = pltpu.async_copy(
            pe_hbm.at[idx_v.at[pl.ds(0, bpc)]], prow_v, gsem
        )
        oridx = jnp.bitwise_or(lanes, 1)[:, None]
        andidx = jnp.bitwise_and(lanes, ~1)[:, None]
        signv = jnp.where(jnp.bitwise_and(lanes, 1) == 0, 1.0, -1.0).astype(
            jnp.float32
        )
        def run_pqprep():
            pq_gather.wait()

            @plsc.parallel_loop(0, ngrp)
            def _pqprep(g):
                g0 = g * _LANES
                for b2 in range(bpc):
                    v = prow_v[b2, pl.ds(g0, _LANES)]
                    pq_v[pl.ds(b2 * 2 * D + g0, _LANES)] = vperm(v, oridx)
                    pq_v[pl.ds(b2 * 2 * D + D + g0, _LANES)] = (
                        vperm(v, andidx) * signv
                    )

        def compute(pt, po):
            tb = tbuf.at[pt]

            @plsc.parallel_loop(0, ngrp)
            def gbody(g):
                g0 = g * _LANES
                pq = [
                    (
                        pq_v[pl.ds(b2 * 2 * D + g0, _LANES)],
                        pq_v[pl.ds(b2 * 2 * D + D + g0, _LANES)],
                    )
                    for b2 in range(bpc)
                ]
                for i in range(_CH):
                    t = tb[i, pl.ds(g0, _LANES)]
                    tsw = vperm(t, swapidx)
                    for b2 in range(bpc):
                        p, q = pq[b2]
                        obuf[b2, po, i, pl.ds(g0, _LANES)] = t * p + tsw * q

        ld = [None] * nch
        st = [[None] * nch for _ in range(bpc)]

        def issue_ld(j):
            pt = j % _NBT
            ld[j] = pltpu.async_copy(
                pe_hbm.at[pl.ds(trow + j * _CH, _CH), :], tbuf.at[pt], ld_sems[pt]
            )

        for j in range(min(2, nch)):
            issue_ld(j)
        run_pqprep()
        for j in range(nch):
            pt = j % _NBT
            po = j % 2
            ld[j].wait()
            if j + 2 < nch:
                issue_ld(j + 2)
            if j >= 2:
                for b2 in range(bpc):
                    st[b2][j - 2].wait()
            compute(pt, po)
            for b2 in range(bpc):
                st[b2][j] = pltpu.async_copy(
                    obuf.at[b2, po],
                    out_hbm.at[pl.ds(orow[b2] + j * _CH, _CH), :],
                    st_sems[b2 * 2 + po],
                )
        for j in range(max(0, nch - 2), nch):
            for b2 in range(bpc):
                st[b2][j].wait()

    return pl.kernel(
        body,
        out_type=jax.ShapeDtypeStruct((B * S, D), jnp.float32),
        mesh=mesh,
        scratch_types=[
            pltpu.VMEM((B,), jnp.int32),
            pltpu.VMEM((_LANES,), jnp.int32),
            pltpu.VMEM((B, D), jnp.float32),
            pltpu.VMEM((B * 2 * D,), jnp.float32),
            pltpu.VMEM((_NBT, _CH, D), jnp.float32),
            pltpu.VMEM((B, 2, _CH, D), jnp.float32),
        ]
        + [pltpu.SemaphoreType.DMA] * (1 + _NBT + 2 * B),
        compiler_params=pltpu.CompilerParams(needs_layout_passes=False),
    )(pe2d, offset)


def kernel(x, offset, pe):
    B, S, D = x.shape
    scale = math.sqrt(D)
    pe2d = pe[0]
    pos_emb = _pos_emb_sc(pe2d, offset, B, S, D).reshape(B, S, D)
    x_scaled = _scale_tc(x, scale)
    return (x_scaled, pos_emb)
